# baseline (speedup 1.0000x reference)
import functools

import jax
import jax.numpy as jnp
from jax import lax
from jax.experimental import pallas as pl
from jax.experimental.pallas import tpu as pltpu
from jax.experimental.pallas import tpu_sc as plsc

B = 16384
D = 32
V = 1_000_000
NC = 2
NS = 16
NW = NC * NS
BPW = B // NW
CHUNKS = BPW // 16

BANDS = D // 8
TILES = V // 128
CW = 12
BW = CW * 128
NCHUNK = TILES // CW
TAIL0 = NCHUNK * BW
TAILW = V - TAIL0
MAIN_ITEMS = 2 * BANDS * NCHUNK
BLOCK_BYTES = 8 * BW * 4
ROW_BYTES = BW * 4


def _detile(ut, it):
    mesh = plsc.VectorSubcoreMesh(core_axis_name="c", subcore_axis_name="s")

    @functools.partial(
        pl.kernel,
        mesh=mesh,
        out_type=(jax.ShapeDtypeStruct((D * V,), jnp.float32),
                  jax.ShapeDtypeStruct((D * V,), jnp.float32)),
        compiler_params=pltpu.CompilerParams(use_tc_tiling_on_sc=True),
        scratch_types=[
            pltpu.VMEM((2, 8, BW), jnp.float32),
            pltpu.VMEM((8, TAILW), jnp.float32),
            pltpu.SemaphoreType.DMA,
            pltpu.SemaphoreType.DMA,
            pltpu.SemaphoreType.DMA,
            pltpu.SemaphoreType.DMA,
        ],
    )
    def run(ut_hbm, it_hbm, ou_hbm, oi_hbm,
            buf_v, tbuf_v, rsem0, rsem1, wsem0, wsem1):
        wid = lax.axis_index("s") * NC + lax.axis_index("c")

        def decode(m):
            tbl = m // (BANDS * NCHUNK)
            rem = m % (BANDS * NCHUNK)
            band = rem // NCHUNK
            col0 = (rem % NCHUNK) * BW
            return tbl, band, col0

        def start_read(m, slot, rsem):
            tbl, band, col0 = decode(m)

            @pl.when(tbl == 0)
            def _():
                pltpu.make_async_copy(
                    ut_hbm.at[pl.ds(band * 8, 8), pl.ds(col0, BW)],
                    buf_v.at[slot], rsem).start()

            @pl.when(tbl == 1)
            def _():
                pltpu.make_async_copy(
                    it_hbm.at[pl.ds(band * 8, 8), pl.ds(col0, BW)],
                    buf_v.at[slot], rsem).start()

        def wait_read(slot, rsem):
            pltpu.make_async_copy(
                ut_hbm.at[pl.ds(0, 8), pl.ds(0, BW)],
                buf_v.at[slot], rsem).wait()

        def start_writes(m, slot, wsem):
            tbl, band, col0 = decode(m)
            for r in range(8):
                off = (band * 8 + r) * V + col0

                @pl.when(tbl == 0)
                def _():
                    pltpu.make_async_copy(
                        buf_v.at[slot, r], ou_hbm.at[pl.ds(off, BW)],
                        wsem).start()

                @pl.when(tbl == 1)
                def _():
                    pltpu.make_async_copy(
                        buf_v.at[slot, r], oi_hbm.at[pl.ds(off, BW)],
                        wsem).start()

        def wait_writes(slot, wsem):
            for r in range(8):
                pltpu.make_async_copy(
                    buf_v.at[slot, r], ou_hbm.at[pl.ds(r * BW, BW)],
                    wsem).wait()

        n_pairs = (MAIN_ITEMS // NW + 1) // 2 + 1

        @pl.when(wid < MAIN_ITEMS)
        def _():
            start_read(wid, 0, rsem0)

        def body(i, carry):
            m0 = wid + (2 * i) * NW
            m1 = m0 + NW
            mn = m0 + 2 * NW

            @pl.when(m1 < MAIN_ITEMS)
            def _():
                start_read(m1, 1, rsem1)

            @pl.when(m0 < MAIN_ITEMS)
            def _():
                wait_read(0, rsem0)
                start_writes(m0, 0, wsem0)

            @pl.when(m1 < MAIN_ITEMS)
            def _():
                wait_read(1, rsem1)
                start_writes(m1, 1, wsem1)

            @pl.when(m0 < MAIN_ITEMS)
            def _():
                wait_writes(0, wsem0)

            @pl.when(m1 < MAIN_ITEMS)
            def _():
                wait_writes(1, wsem1)

            @pl.when(mn < MAIN_ITEMS)
            def _():
                start_read(mn, 0, rsem0)

            return carry

        lax.fori_loop(0, n_pairs, body, 0)

        @pl.when(wid < 2 * BANDS)
        def _():
            tbl = wid // BANDS
            band = wid % BANDS

            @pl.when(tbl == 0)
            def _():
                pltpu.sync_copy(
                    ut_hbm.at[pl.ds(band * 8, 8), pl.ds(TAIL0, TAILW)],
                    tbuf_v)

            @pl.when(tbl == 1)
            def _():
                pltpu.sync_copy(
                    it_hbm.at[pl.ds(band * 8, 8), pl.ds(TAIL0, TAILW)],
                    tbuf_v)

            for r in range(8):
                off = (band * 8 + r) * V + TAIL0

                @pl.when(tbl == 0)
                def _():
                    pltpu.sync_copy(tbuf_v.at[r], ou_hbm.at[pl.ds(off, TAILW)])

                @pl.when(tbl == 1)
                def _():
                    pltpu.sync_copy(tbuf_v.at[r], oi_hbm.at[pl.ds(off, TAILW)])

    return run(ut, it)


def kernel(user_id, item_id, user_table, item_table):
    u_lin = jnp.reshape(user_table.T, (D * V,))
    i_lin = jnp.reshape(item_table.T, (D * V,))

    mesh = plsc.VectorSubcoreMesh(core_axis_name="c", subcore_axis_name="s")

    @functools.partial(
        pl.kernel,
        mesh=mesh,
        out_type=jax.ShapeDtypeStruct((B,), jnp.float32),
        compiler_params=pltpu.CompilerParams(use_tc_tiling_on_sc=False),
        scratch_types=[
            pltpu.VMEM((BPW,), jnp.int32),
            pltpu.VMEM((BPW,), jnp.int32),
            pltpu.VMEM((D, BPW), jnp.float32),
            pltpu.VMEM((D, BPW), jnp.float32),
            pltpu.VMEM((BPW,), jnp.float32),
            pltpu.SemaphoreType.DMA,
        ],
    )
    def run(uid_hbm, iid_hbm, ul_hbm, il_hbm, out_hbm,
            uidx_v, iidx_v, ucols_v, icols_v, out_v, sem):
        wid = lax.axis_index("s") * NC + lax.axis_index("c")
        base = wid * BPW

        pltpu.sync_copy(uid_hbm.at[pl.ds(base, BPW)], uidx_v)
        pltpu.sync_copy(iid_hbm.at[pl.ds(base, BPW)], iidx_v)

        copies = []
        for d in range(D):
            copies.append(pltpu.async_copy(
                ul_hbm.at[pl.ds(d * V, V)].at[uidx_v], ucols_v.at[d], sem))
            copies.append(pltpu.async_copy(
                il_hbm.at[pl.ds(d * V, V)].at[iidx_v], icols_v.at[d], sem))
        for c in copies:
            c.wait()

        def chunk_body(c, carry):
            off = c * 16
            acc = jnp.zeros((16,), jnp.float32)
            for d in range(D):
                acc = acc + (ucols_v[d, pl.ds(off, 16)]
                             * icols_v[d, pl.ds(off, 16)])
            out_v[pl.ds(off, 16)] = acc
            return carry

        lax.fori_loop(0, CHUNKS, chunk_body, 0)

        pltpu.sync_copy(out_v, out_hbm.at[pl.ds(base, BPW)])

    return run(user_id, item_id, u_lin, i_lin)
